# Initial kernel scaffold; baseline (speedup 1.0000x reference)
import jax
import jax.numpy as jnp
from jax.experimental import pallas as pl


def kernel(h, edge_index, Wk, bk, Wq, bq, Wv, bv, Wa, ba, rel_pri, rel_att, rel_msg, skip):
    raise NotImplementedError("write your pallas kernel here")


# speedup vs baseline: 1.0321x; 1.0321x over previous
import math
import functools

import jax
import jax.numpy as jnp
from jax import lax
from jax.experimental import pallas as pl
from jax.experimental.pallas import tpu as pltpu

N = 10000
E = 160000
D = 256
H = 8
DK = 32
ROW_BLK = 1000


def _combine_kernel(Wk_ref, bk_ref, Wv_ref, bv_ref, ra_ref, rm_ref,
                    Mk_ref, bke_ref, Mv_ref, bve_ref):
    for h in range(H):
        wk = Wk_ref[pl.ds(h * DK, DK), :]
        wv = Wv_ref[pl.ds(h * DK, DK), :]
        ra = ra_ref[h]
        rm = rm_ref[h]
        dn = (((0,), (0,)), ((), ()))
        Mk_ref[:, pl.ds(h * DK, DK)] = lax.dot_general(
            wk, ra, dn, preferred_element_type=jnp.float32)
        Mv_ref[:, pl.ds(h * DK, DK)] = lax.dot_general(
            wv, rm, dn, preferred_element_type=jnp.float32)
        bke_ref[0, pl.ds(h * DK, DK)] = (
            bk_ref[0, pl.ds(h * DK, DK)] @ ra)
        bve_ref[0, pl.ds(h * DK, DK)] = (
            bv_ref[0, pl.ds(h * DK, DK)] @ rm)


def _combine(Wk, bk, Wv, bv, rel_att, rel_msg):
    out_shapes = (
        jax.ShapeDtypeStruct((D, D), jnp.float32),
        jax.ShapeDtypeStruct((1, D), jnp.float32),
        jax.ShapeDtypeStruct((D, D), jnp.float32),
        jax.ShapeDtypeStruct((1, D), jnp.float32),
    )
    return pl.pallas_call(
        _combine_kernel,
        out_shape=out_shapes,
    )(Wk, bk.reshape(1, D), Wv, bv.reshape(1, D), rel_att, rel_msg)


def _qkv_kernel(h_ref, Wq_ref, bq_ref, Mk_ref, bke_ref, Mv_ref, bve_ref,
                q_ref, k_ref, v_ref):
    x = h_ref[...]
    q_ref[...] = jnp.dot(x, Wq_ref[...].T,
                         preferred_element_type=jnp.float32) + bq_ref[...]
    k_ref[...] = jnp.dot(x, Mk_ref[...],
                         preferred_element_type=jnp.float32) + bke_ref[...]
    v_ref[...] = jnp.dot(x, Mv_ref[...],
                         preferred_element_type=jnp.float32) + bve_ref[...]


def _qkv(h, Wq, bq, Mk, bke, Mv, bve):
    grid = (N // ROW_BLK,)
    blk = lambda i: (i, 0)
    w_spec = pl.BlockSpec((D, D), lambda i: (0, 0))
    b_spec = pl.BlockSpec((1, D), lambda i: (0, 0))
    out_spec = pl.BlockSpec((ROW_BLK, D), blk)
    return pl.pallas_call(
        _qkv_kernel,
        grid=grid,
        in_specs=[pl.BlockSpec((ROW_BLK, D), blk),
                  w_spec, b_spec, w_spec, b_spec, w_spec, b_spec],
        out_specs=(out_spec, out_spec, out_spec),
        out_shape=(jax.ShapeDtypeStruct((N, D), jnp.float32),) * 3,
    )(h, Wq, bq.reshape(1, D), Mk, bke, Mv, bve)


def _final_kernel(agg_ref, h_ref, Wa_ref, ba_ref, skip_ref, out_ref):
    alpha = jax.nn.sigmoid(skip_ref[0])
    trans = jnp.dot(agg_ref[...], Wa_ref[...].T,
                    preferred_element_type=jnp.float32) + ba_ref[...]
    out_ref[...] = trans * alpha + h_ref[...] * (1.0 - alpha)


def _final(agg, h, Wa, ba, skip):
    grid = (N // ROW_BLK,)
    blk = lambda i: (i, 0)
    return pl.pallas_call(
        _final_kernel,
        grid=grid,
        in_specs=[pl.BlockSpec((ROW_BLK, D), blk),
                  pl.BlockSpec((ROW_BLK, D), blk),
                  pl.BlockSpec((D, D), lambda i: (0, 0)),
                  pl.BlockSpec((1, D), lambda i: (0, 0)),
                  pl.BlockSpec(memory_space=pltpu.SMEM)],
        out_specs=pl.BlockSpec((ROW_BLK, D), blk),
        out_shape=jax.ShapeDtypeStruct((N, D), jnp.float32),
    )(agg, h, Wa, ba.reshape(1, D), skip)


def kernel(h, edge_index, Wk, bk, Wq, bq, Wv, bv, Wa, ba, rel_pri, rel_att,
           rel_msg, skip):
    src = edge_index[0]
    dst = edge_index[1]
    Mk, bke, Mv, bve = _combine(Wk, bk, Wv, bv, rel_att, rel_msg)
    q, k, v = _qkv(h, Wq, bq, Mk, bke, Mv, bve)
    qh = q.reshape(N, H, DK)
    kh = k.reshape(N, H, DK)
    vh = v.reshape(N, H, DK)
    t = jnp.sum(qh[dst] * kh[src], axis=-1)
    attn = t * rel_pri / math.sqrt(DK)
    ex = jnp.exp(attn)
    denom = jax.ops.segment_sum(ex, dst, num_segments=N)
    attn_n = ex / (denom[dst] + 1e-16)
    agg = jax.ops.segment_sum(vh[src] * attn_n[:, :, None], dst,
                              num_segments=N).reshape(N, D)
    return _final(agg, h, Wa, ba, skip)


# speedup vs baseline: 31.8767x; 30.8852x over previous
import math
import functools

import jax
import jax.numpy as jnp
from jax import lax
from jax.experimental import pallas as pl
from jax.experimental.pallas import tpu as pltpu
from jax.experimental.pallas import tpu_sc as plsc

N = 10000
E = 160000
D = 256
H = 8
DK = 32
ROW_BLK = 1000

NS = 16
NC = 2
EPT = E // NS
C = 80
NCHUNK = EPT // C
RCH = 80
NRC = N // RCH
HW = 128


def _combine_kernel(Wk_ref, bk_ref, Wv_ref, bv_ref, ra_ref, rm_ref,
                    Mk_ref, bke_ref, Mv_ref, bve_ref):
    for h in range(H):
        wk = Wk_ref[pl.ds(h * DK, DK), :]
        wv = Wv_ref[pl.ds(h * DK, DK), :]
        ra = ra_ref[h]
        rm = rm_ref[h]
        dn = (((0,), (0,)), ((), ()))
        Mk_ref[:, pl.ds(h * DK, DK)] = lax.dot_general(
            wk, ra, dn, preferred_element_type=jnp.float32)
        Mv_ref[:, pl.ds(h * DK, DK)] = lax.dot_general(
            wv, rm, dn, preferred_element_type=jnp.float32)
        bke_ref[0, pl.ds(h * DK, DK)] = bk_ref[0, pl.ds(h * DK, DK)] @ ra
        bve_ref[0, pl.ds(h * DK, DK)] = bv_ref[0, pl.ds(h * DK, DK)] @ rm


def _combine(Wk, bk, Wv, bv, rel_att, rel_msg):
    out_shapes = (
        jax.ShapeDtypeStruct((D, D), jnp.float32),
        jax.ShapeDtypeStruct((1, D), jnp.float32),
        jax.ShapeDtypeStruct((D, D), jnp.float32),
        jax.ShapeDtypeStruct((1, D), jnp.float32),
    )
    return pl.pallas_call(
        _combine_kernel,
        out_shape=out_shapes,
    )(Wk, bk.reshape(1, D), Wv, bv.reshape(1, D), rel_att, rel_msg)


def _qkv_kernel(h_ref, Wq_ref, bq_ref, Mk_ref, bke_ref, Mv_ref, bve_ref,
                q_ref, k_ref, v_ref):
    x = h_ref[...]
    q = jnp.dot(x, Wq_ref[...].T,
                preferred_element_type=jnp.float32) + bq_ref[...]
    k = jnp.dot(x, Mk_ref[...],
                preferred_element_type=jnp.float32) + bke_ref[...]
    v = jnp.dot(x, Mv_ref[...],
                preferred_element_type=jnp.float32) + bve_ref[...]
    q_ref[0] = q[:, :HW]
    q_ref[1] = q[:, HW:]
    k_ref[0] = k[:, :HW]
    k_ref[1] = k[:, HW:]
    v_ref[0] = v[:, :HW]
    v_ref[1] = v[:, HW:]


def _qkv(h, Wq, bq, Mk, bke, Mv, bve):
    grid = (N // ROW_BLK,)
    w_spec = pl.BlockSpec((D, D), lambda i: (0, 0))
    b_spec = pl.BlockSpec((1, D), lambda i: (0, 0))
    out_spec = pl.BlockSpec((NC, ROW_BLK, HW), lambda i: (0, i, 0))
    return pl.pallas_call(
        _qkv_kernel,
        grid=grid,
        in_specs=[pl.BlockSpec((ROW_BLK, D), lambda i: (i, 0)),
                  w_spec, b_spec, w_spec, b_spec, w_spec, b_spec],
        out_specs=(out_spec, out_spec, out_spec),
        out_shape=(jax.ShapeDtypeStruct((NC, N, HW), jnp.float32),) * 3,
    )(h, Wq, bq.reshape(1, D), Mk, bke, Mv, bve)


def _dg(x, idx):
    return x.at[idx].get(mode="promise_in_bounds")


def _edge_body(qt, kt, vt, srch, dsth, scale_h, out,
               src_v, dst_v, srco_v, dsto_v, idx4_v, qb, kb, vb, wvb, exb,
               scale_v, sh_agg, sh_den, sem):
    c = lax.axis_index("c")
    s = lax.axis_index("s")
    lanes = lax.iota(jnp.int32, 16)
    zeros16 = jnp.zeros((16,), jnp.float32)

    r8 = (lanes + 8) & 15
    p4 = (lanes & 8) | ((lanes + 4) & 7)
    p2 = (lanes & 8) | ((lanes + 2) & 7)
    p1 = (lanes & 8) | ((lanes + 1) & 7)
    i8 = (lanes & 1) * 8
    l4 = (lanes - 4) & 15
    l8 = (lanes - 8) & 15
    l12 = (lanes - 12) & 15

    pltpu.sync_copy(scale_h.at[c], scale_v)

    @pl.loop(0, RCH)
    def _zero_buf(r):
        for i in range(8):
            qb.at[r][pl.ds(i * 16, 16)] = zeros16

    @pl.loop(0, C * 4 // 16)
    def _zero_den(t):
        exb[pl.ds(16 * t, 16)] = zeros16

    for t in range(8):
        cc = s + NS * t

        @pl.when(cc < NRC)
        def _():
            pltpu.sync_copy(qb, sh_agg.at[pl.ds(cc * RCH, RCH)])
            pltpu.sync_copy(exb, sh_den.at[pl.ds(cc * RCH * 4, RCH * 4)])

    plsc.subcore_barrier()

    @pl.loop(0, NCHUNK)
    def _chunk(ch):
        base = s * EPT + ch * C
        pltpu.sync_copy(srch.at[pl.ds(base, C)], src_v)
        pltpu.sync_copy(dsth.at[pl.ds(base, C)], dst_v)
        off = c * N
        for g in range(C // 16):
            sl = pl.ds(g * 16, 16)
            srco_v[sl] = src_v[sl] + off
            dsto_v[sl] = dst_v[sl] + off
        cp1 = pltpu.async_copy(qt.at[dsto_v], qb, sem)
        cp2 = pltpu.async_copy(kt.at[srco_v], kb, sem)
        cp3 = pltpu.async_copy(vt.at[srco_v], vb, sem)
        cp1.wait()
        cp2.wait()
        cp3.wait()
        for g4 in range(C // 16):
            dvv = dst_v[pl.ds(g4 * 16, 16)]
            for q in range(4):
                d4 = _dg(dvv, (lanes >> 2) + 4 * q)
                idx4_v[pl.ds(g4 * 64 + q * 16, 16)] = d4 * 4 + (lanes & 3)
        sv = scale_v[pl.ds(0, 16)]

        def _attn(e):
            qe = qb.at[e]
            ke = kb.at[e]
            ss = []
            for hh in range(4):
                pa = qe[pl.ds(hh * 32, 16)] * ke[pl.ds(hh * 32, 16)]
                pb = qe[pl.ds(hh * 32 + 16, 16)] * ke[pl.ds(hh * 32 + 16, 16)]
                ss.append(pa + pb)
            t0 = ss[0] + _dg(ss[0], r8)
            t1 = ss[1] + _dg(ss[1], r8)
            t2 = ss[2] + _dg(ss[2], r8)
            t3 = ss[3] + _dg(ss[3], r8)
            u01 = jnp.where(lanes < 8, t0, _dg(t1, r8))
            u23 = jnp.where(lanes < 8, t2, _dg(t3, r8))
            u01 = u01 + _dg(u01, p4)
            u23 = u23 + _dg(u23, p4)
            u01 = u01 + _dg(u01, p2)
            u23 = u23 + _dg(u23, p2)
            u01 = u01 + _dg(u01, p1)
            u23 = u23 + _dg(u23, p1)
            w = jnp.where(lanes < 2, _dg(u01, i8), _dg(u23, i8))
            ex = jnp.exp(w * sv)
            ve = vb.at[e]
            we = wvb.at[e]
            for hh in range(4):
                wb = _dg(ex, jnp.full((16,), hh, jnp.int32))
                for j in range(2):
                    sl = pl.ds(hh * 32 + j * 16, 16)
                    we[sl] = ve[sl] * wb
            return ex

        @pl.loop(0, C // 4)
        def _quad(p):
            ex0 = _attn(4 * p)
            ex1 = _attn(4 * p + 1)
            ex2 = _attn(4 * p + 2)
            ex3 = _attn(4 * p + 3)
            packed = jnp.where(
                lanes < 4, ex0,
                jnp.where(lanes < 8, _dg(ex1, l4),
                          jnp.where(lanes < 12, _dg(ex2, l8),
                                    _dg(ex3, l12))))
            exb[pl.ds(16 * p, 16)] = packed

        pltpu.sync_copy(wvb, sh_agg.at[dst_v], add=True)
        pltpu.sync_copy(exb, sh_den.at[idx4_v], add=True)

    plsc.subcore_barrier()

    for t in range(8):
        cc = s + NS * t

        @pl.when(cc < NRC)
        def _():
            r0 = cc * RCH
            pltpu.sync_copy(sh_agg.at[pl.ds(r0, RCH)], qb)
            pltpu.sync_copy(sh_den.at[pl.ds(r0 * 4, RCH * 4)], exb)

            @pl.loop(0, RCH // 4)
            def _rowquad(p):
                den = exb[pl.ds(16 * p, 16)]
                rcp = 1.0 / (den + 1e-16)
                for rr in range(4):
                    zr = qb.at[4 * p + rr]
                    for hh in range(4):
                        rb = _dg(rcp, jnp.full((16,), rr * 4 + hh,
                                               jnp.int32))
                        for j in range(2):
                            sl = pl.ds(hh * 32 + j * 16, 16)
                            zr[sl] = zr[sl] * rb

            pltpu.sync_copy(qb, out.at[pl.ds(c * N + r0, RCH)])


def _edge_sc(qt, kt, vt, src, dst, scale):
    mesh = plsc.VectorSubcoreMesh(core_axis_name="c", subcore_axis_name="s")

    f = pl.kernel(
        _edge_body,
        out_type=jax.ShapeDtypeStruct((NC * N, HW), jnp.float32),
        mesh=mesh,
        scratch_types=[
            pltpu.VMEM((C,), jnp.int32),
            pltpu.VMEM((C,), jnp.int32),
            pltpu.VMEM((C,), jnp.int32),
            pltpu.VMEM((C,), jnp.int32),
            pltpu.VMEM((C * 4,), jnp.int32),
            pltpu.VMEM((C, HW), jnp.float32),
            pltpu.VMEM((C, HW), jnp.float32),
            pltpu.VMEM((C, HW), jnp.float32),
            pltpu.VMEM((C, HW), jnp.float32),
            pltpu.VMEM((C * 4,), jnp.float32),
            pltpu.VMEM((16,), jnp.float32),
            pltpu.VMEM_SHARED((N, HW), jnp.float32),
            pltpu.VMEM_SHARED((N * 4,), jnp.float32),
            pltpu.SemaphoreType.DMA,
        ],
    )
    return f(qt.reshape(NC * N, HW), kt.reshape(NC * N, HW),
             vt.reshape(NC * N, HW), src, dst, scale)


def _final_kernel(agg_ref, h_ref, Wa_ref, ba_ref, skip_ref, out_ref):
    alpha = jax.nn.sigmoid(skip_ref[0])
    dn = (((1,), (1,)), ((), ()))
    trans = lax.dot_general(agg_ref[0], Wa_ref[:, :HW], dn,
                            preferred_element_type=jnp.float32)
    trans = trans + lax.dot_general(agg_ref[1], Wa_ref[:, HW:], dn,
                                    preferred_element_type=jnp.float32)
    trans = trans + ba_ref[...]
    out_ref[...] = trans * alpha + h_ref[...] * (1.0 - alpha)


def _final(agg2, h, Wa, ba, skip):
    grid = (N // ROW_BLK,)
    return pl.pallas_call(
        _final_kernel,
        grid=grid,
        in_specs=[pl.BlockSpec((NC, ROW_BLK, HW), lambda i: (0, i, 0)),
                  pl.BlockSpec((ROW_BLK, D), lambda i: (i, 0)),
                  pl.BlockSpec((D, D), lambda i: (0, 0)),
                  pl.BlockSpec((1, D), lambda i: (0, 0)),
                  pl.BlockSpec(memory_space=pltpu.SMEM)],
        out_specs=pl.BlockSpec((ROW_BLK, D), lambda i: (i, 0)),
        out_shape=jax.ShapeDtypeStruct((N, D), jnp.float32),
    )(agg2, h, Wa, ba.reshape(1, D), skip)


def kernel(h, edge_index, Wk, bk, Wq, bq, Wv, bv, Wa, ba, rel_pri, rel_att,
           rel_msg, skip):
    src = edge_index[0]
    dst = edge_index[1]
    Mk, bke, Mv, bve = _combine(Wk, bk, Wv, bv, rel_att, rel_msg)
    qt, kt, vt = _qkv(h, Wq, bq, Mk, bke, Mv, bve)
    scale = jnp.zeros((NC, 16), jnp.float32)
    scale = scale.at[:, :4].set(
        (rel_pri / math.sqrt(DK)).reshape(NC, 4))
    agg = _edge_sc(qt, kt, vt, src, dst, scale)
    return _final(agg.reshape(NC, N, HW), h, Wa, ba, skip)


# speedup vs baseline: 44.2223x; 1.3873x over previous
import math
import functools

import jax
import jax.numpy as jnp
from jax import lax
from jax.experimental import pallas as pl
from jax.experimental.pallas import tpu as pltpu
from jax.experimental.pallas import tpu_sc as plsc

N = 10000
E = 160000
D = 256
H = 8
DK = 32
ROW_BLK = 1000

NS = 16
NC = 2
EPT = E // NS
C = 80
NCHUNK = EPT // C
RCH = 80
NRC = N // RCH
HW = 128


def _combine_kernel(Wk_ref, bk_ref, Wv_ref, bv_ref, ra_ref, rm_ref,
                    Mk_ref, bke_ref, Mv_ref, bve_ref):
    for h in range(H):
        wk = Wk_ref[pl.ds(h * DK, DK), :]
        wv = Wv_ref[pl.ds(h * DK, DK), :]
        ra = ra_ref[h]
        rm = rm_ref[h]
        dn = (((0,), (0,)), ((), ()))
        Mk_ref[:, pl.ds(h * DK, DK)] = lax.dot_general(
            wk, ra, dn, preferred_element_type=jnp.float32)
        Mv_ref[:, pl.ds(h * DK, DK)] = lax.dot_general(
            wv, rm, dn, preferred_element_type=jnp.float32)
        bke_ref[0, pl.ds(h * DK, DK)] = bk_ref[0, pl.ds(h * DK, DK)] @ ra
        bve_ref[0, pl.ds(h * DK, DK)] = bv_ref[0, pl.ds(h * DK, DK)] @ rm


def _combine(Wk, bk, Wv, bv, rel_att, rel_msg):
    out_shapes = (
        jax.ShapeDtypeStruct((D, D), jnp.float32),
        jax.ShapeDtypeStruct((1, D), jnp.float32),
        jax.ShapeDtypeStruct((D, D), jnp.float32),
        jax.ShapeDtypeStruct((1, D), jnp.float32),
    )
    return pl.pallas_call(
        _combine_kernel,
        out_shape=out_shapes,
    )(Wk, bk.reshape(1, D), Wv, bv.reshape(1, D), rel_att, rel_msg)


def _qkv_kernel(h_ref, Wq_ref, bq_ref, Mk_ref, bke_ref, Mv_ref, bve_ref,
                q_ref, k_ref, v_ref):
    x = h_ref[...]
    q = jnp.dot(x, Wq_ref[...].T,
                preferred_element_type=jnp.float32) + bq_ref[...]
    k = jnp.dot(x, Mk_ref[...],
                preferred_element_type=jnp.float32) + bke_ref[...]
    v = jnp.dot(x, Mv_ref[...],
                preferred_element_type=jnp.float32) + bve_ref[...]
    q_ref[0] = q[:, :HW]
    q_ref[1] = q[:, HW:]
    k_ref[0] = k[:, :HW]
    k_ref[1] = k[:, HW:]
    v_ref[0] = v[:, :HW]
    v_ref[1] = v[:, HW:]


def _qkv(h, Wq, bq, Mk, bke, Mv, bve):
    grid = (N // ROW_BLK,)
    w_spec = pl.BlockSpec((D, D), lambda i: (0, 0))
    b_spec = pl.BlockSpec((1, D), lambda i: (0, 0))
    out_spec = pl.BlockSpec((NC, ROW_BLK, HW), lambda i: (0, i, 0))
    return pl.pallas_call(
        _qkv_kernel,
        grid=grid,
        in_specs=[pl.BlockSpec((ROW_BLK, D), lambda i: (i, 0)),
                  w_spec, b_spec, w_spec, b_spec, w_spec, b_spec],
        out_specs=(out_spec, out_spec, out_spec),
        out_shape=(jax.ShapeDtypeStruct((NC, N, HW), jnp.float32),) * 3,
    )(h, Wq, bq.reshape(1, D), Mk, bke, Mv, bve)


def _dg(x, idx):
    return x.at[idx].get(mode="promise_in_bounds")


def _edge_body(qt, kt, vt, srch, dsth, scale_h, out,
               src_v0, dst_v0, dsto_v0, src_v1, dst_v1, dsto_v1,
               qb0, qb1, kb, vb, idx4_v, exb, scale_v,
               sh_agg, sh_den, semq0, semq1, semk, semv):
    c = lax.axis_index("c")
    s = lax.axis_index("s")
    lanes = lax.iota(jnp.int32, 16)
    zeros16 = jnp.zeros((16,), jnp.float32)

    src_v = [src_v0, src_v1]
    dst_v = [dst_v0, dst_v1]
    dsto_v = [dsto_v0, dsto_v1]
    qb = [qb0, qb1]
    semq = [semq0, semq1]

    r8 = (lanes + 8) & 15
    p4 = (lanes & 8) | ((lanes + 4) & 7)
    p2 = (lanes & 8) | ((lanes + 2) & 7)
    p1 = (lanes & 8) | ((lanes + 1) & 7)
    i8 = (lanes & 1) * 8
    l4 = (lanes - 4) & 15
    l8 = (lanes - 8) & 15
    l12 = (lanes - 12) & 15

    pltpu.sync_copy(scale_h.at[c], scale_v)
    off = c * N

    @pl.loop(0, RCH)
    def _zero_buf(r):
        for i in range(8):
            qb0.at[r][pl.ds(i * 16, 16)] = zeros16

    @pl.loop(0, C * 4 // 16)
    def _zero_den(t):
        exb[pl.ds(16 * t, 16)] = zeros16

    for t in range(8):
        cc = s + NS * t

        @pl.when(cc < NRC)
        def _():
            pltpu.sync_copy(qb0, sh_agg.at[pl.ds(cc * RCH, RCH)])
            pltpu.sync_copy(exb, sh_den.at[pl.ds(cc * RCH * 4, RCH * 4)])

    plsc.subcore_barrier()


    def stage(b, ch):
        base = s * EPT + ch * C
        pltpu.sync_copy(srch.at[pl.ds(base, C)], src_v[b])
        pltpu.sync_copy(dsth.at[pl.ds(base, C)], dst_v[b])
        for g in range(C // 16):
            sl = pl.ds(g * 16, 16)
            src_v[b][sl] = src_v[b][sl] + off
            dsto_v[b][sl] = dst_v[b][sl] + off
        pltpu.async_copy(qt.at[dsto_v[b]], qb[b], semq[b])

    def fire_k(b):
        pltpu.async_copy(kt.at[src_v[b]], kb, semk)

    def fire_v(b):
        pltpu.async_copy(vt.at[src_v[b]], vb, semv)

    def _attn(b, e):
        qe = qb[b].at[e]
        ke = kb.at[e]
        ss = []
        for hh in range(4):
            pa = qe[pl.ds(hh * 32, 16)] * ke[pl.ds(hh * 32, 16)]
            pb = qe[pl.ds(hh * 32 + 16, 16)] * ke[pl.ds(hh * 32 + 16, 16)]
            ss.append(pa + pb)
        t0 = ss[0] + _dg(ss[0], r8)
        t1 = ss[1] + _dg(ss[1], r8)
        t2 = ss[2] + _dg(ss[2], r8)
        t3 = ss[3] + _dg(ss[3], r8)
        u01 = jnp.where(lanes < 8, t0, _dg(t1, r8))
        u23 = jnp.where(lanes < 8, t2, _dg(t3, r8))
        u01 = u01 + _dg(u01, p4)
        u23 = u23 + _dg(u23, p4)
        u01 = u01 + _dg(u01, p2)
        u23 = u23 + _dg(u23, p2)
        u01 = u01 + _dg(u01, p1)
        u23 = u23 + _dg(u23, p1)
        w = jnp.where(lanes < 2, _dg(u01, i8), _dg(u23, i8))
        sv = scale_v[pl.ds(0, 16)]
        return jnp.exp(w * sv)

    def dot_phase(b):
        @pl.loop(0, C // 4)
        def _quad(p):
            ex0 = _attn(b, 4 * p)
            ex1 = _attn(b, 4 * p + 1)
            ex2 = _attn(b, 4 * p + 2)
            ex3 = _attn(b, 4 * p + 3)
            packed = jnp.where(
                lanes < 4, ex0,
                jnp.where(lanes < 8, _dg(ex1, l4),
                          jnp.where(lanes < 12, _dg(ex2, l8),
                                    _dg(ex3, l12))))
            exb[pl.ds(16 * p, 16)] = packed

    def wv_phase(b):
        @pl.loop(0, C // 4)
        def _quad(p):
            exB = exb[pl.ds(16 * p, 16)]
            for sub in range(4):
                e = 4 * p + sub
                ve = vb.at[e]
                we = qb[b].at[e]
                for hh in range(4):
                    wb = _dg(exB, jnp.full((16,), 4 * sub + hh, jnp.int32))
                    for j in range(2):
                        sl = pl.ds(hh * 32 + j * 16, 16)
                        we[sl] = ve[sl] * wb

    def scatter(b):
        for g4 in range(C // 16):
            dvv = dst_v[b][pl.ds(g4 * 16, 16)]
            for q in range(4):
                d4 = _dg(dvv, (lanes >> 2) + 4 * q)
                idx4_v[pl.ds(g4 * 64 + q * 16, 16)] = d4 * 4 + (lanes & 3)
        pltpu.sync_copy(qb[b], sh_agg.at[dst_v[b]], add=True)
        pltpu.sync_copy(exb, sh_den.at[idx4_v], add=True)

    def substep(b, ch, last):
        if not last:
            stage(1 - b, ch + 1)
        pltpu.make_async_copy(qt.at[dsto_v[b]], qb[b], semq[b]).wait()
        pltpu.make_async_copy(kt.at[src_v[b]], kb, semk).wait()
        dot_phase(b)
        if not last:
            fire_k(1 - b)
        pltpu.make_async_copy(vt.at[src_v[b]], vb, semv).wait()
        wv_phase(b)
        if not last:
            fire_v(1 - b)
        scatter(b)

    stage(0, 0)
    fire_k(0)
    fire_v(0)

    @pl.loop(0, (NCHUNK - 1) // 2)
    def _super(ssi):
        substep(0, 2 * ssi, False)
        substep(1, 2 * ssi + 1, False)

    substep(0, NCHUNK - 1, True)

    plsc.subcore_barrier()

    for t in range(8):
        cc = s + NS * t

        @pl.when(cc < NRC)
        def _():
            r0 = cc * RCH
            pltpu.sync_copy(sh_agg.at[pl.ds(r0, RCH)], qb0)
            pltpu.sync_copy(sh_den.at[pl.ds(r0 * 4, RCH * 4)], exb)

            @pl.loop(0, RCH // 4)
            def _rowquad(p):
                den = exb[pl.ds(16 * p, 16)]
                rcp = 1.0 / (den + 1e-16)
                for rr in range(4):
                    zr = qb0.at[4 * p + rr]
                    for hh in range(4):
                        rb = _dg(rcp, jnp.full((16,), rr * 4 + hh,
                                               jnp.int32))
                        for j in range(2):
                            sl = pl.ds(hh * 32 + j * 16, 16)
                            zr[sl] = zr[sl] * rb

            pltpu.sync_copy(qb0, out.at[pl.ds(c * N + r0, RCH)])


def _edge_sc(qt, kt, vt, src, dst, scale):
    mesh = plsc.VectorSubcoreMesh(core_axis_name="c", subcore_axis_name="s")

    f = pl.kernel(
        _edge_body,
        out_type=jax.ShapeDtypeStruct((NC * N, HW), jnp.float32),
        mesh=mesh,
        scratch_types=(
            [pltpu.VMEM((C,), jnp.int32)] * 6
            + [pltpu.VMEM((C, HW), jnp.float32)] * 4
            + [pltpu.VMEM((C * 4,), jnp.int32),
               pltpu.VMEM((C * 4,), jnp.float32),
               pltpu.VMEM((16,), jnp.float32),
               pltpu.VMEM_SHARED((N, HW), jnp.float32),
               pltpu.VMEM_SHARED((N * 4,), jnp.float32)]
            + [pltpu.SemaphoreType.DMA] * 4
        ),
    )
    return f(qt.reshape(NC * N, HW), kt.reshape(NC * N, HW),
             vt.reshape(NC * N, HW), src, dst, scale)


def _final_kernel(agg_ref, h_ref, Wa_ref, ba_ref, skip_ref, out_ref):
    alpha = jax.nn.sigmoid(skip_ref[0])
    dn = (((1,), (1,)), ((), ()))
    trans = lax.dot_general(agg_ref[0], Wa_ref[:, :HW], dn,
                            preferred_element_type=jnp.float32)
    trans = trans + lax.dot_general(agg_ref[1], Wa_ref[:, HW:], dn,
                                    preferred_element_type=jnp.float32)
    trans = trans + ba_ref[...]
    out_ref[...] = trans * alpha + h_ref[...] * (1.0 - alpha)


def _final(agg2, h, Wa, ba, skip):
    grid = (N // ROW_BLK,)
    return pl.pallas_call(
        _final_kernel,
        grid=grid,
        in_specs=[pl.BlockSpec((NC, ROW_BLK, HW), lambda i: (0, i, 0)),
                  pl.BlockSpec((ROW_BLK, D), lambda i: (i, 0)),
                  pl.BlockSpec((D, D), lambda i: (0, 0)),
                  pl.BlockSpec((1, D), lambda i: (0, 0)),
                  pl.BlockSpec(memory_space=pltpu.SMEM)],
        out_specs=pl.BlockSpec((ROW_BLK, D), lambda i: (i, 0)),
        out_shape=jax.ShapeDtypeStruct((N, D), jnp.float32),
    )(agg2, h, Wa, ba.reshape(1, D), skip)


def kernel(h, edge_index, Wk, bk, Wq, bq, Wv, bv, Wa, ba, rel_pri, rel_att,
           rel_msg, skip):
    src = edge_index[0]
    dst = edge_index[1]
    Mk, bke, Mv, bve = _combine(Wk, bk, Wv, bv, rel_att, rel_msg)
    qt, kt, vt = _qkv(h, Wq, bq, Mk, bke, Mv, bve)
    scale = jnp.zeros((NC, 16), jnp.float32)
    scale = scale.at[:, :4].set(
        (rel_pri / math.sqrt(DK)).reshape(NC, 4))
    agg = _edge_sc(qt, kt, vt, src, dst, scale)
    return _final(agg.reshape(NC, N, HW), h, Wa, ba, skip)


# speedup vs baseline: 45.6285x; 1.0318x over previous
import math
import functools

import jax
import jax.numpy as jnp
from jax import lax
from jax.experimental import pallas as pl
from jax.experimental.pallas import tpu as pltpu
from jax.experimental.pallas import tpu_sc as plsc

N = 10000
E = 160000
D = 256
H = 8
DK = 32
ROW_BLK = 1000

NS = 16
NC = 2
EPT = E // NS
C = 80
NCHUNK = EPT // C
RCH = 80
NRC = N // RCH
HW = 128


def _combine_kernel(Wk_ref, bk_ref, Wv_ref, bv_ref, ra_ref, rm_ref,
                    Mk_ref, bke_ref, Mv_ref, bve_ref):
    for h in range(H):
        wk = Wk_ref[pl.ds(h * DK, DK), :]
        wv = Wv_ref[pl.ds(h * DK, DK), :]
        ra = ra_ref[h]
        rm = rm_ref[h]
        dn = (((0,), (0,)), ((), ()))
        Mk_ref[:, pl.ds(h * DK, DK)] = lax.dot_general(
            wk, ra, dn, preferred_element_type=jnp.float32)
        Mv_ref[:, pl.ds(h * DK, DK)] = lax.dot_general(
            wv, rm, dn, preferred_element_type=jnp.float32)
        bke_ref[0, pl.ds(h * DK, DK)] = bk_ref[0, pl.ds(h * DK, DK)] @ ra
        bve_ref[0, pl.ds(h * DK, DK)] = bv_ref[0, pl.ds(h * DK, DK)] @ rm


def _combine(Wk, bk, Wv, bv, rel_att, rel_msg):
    out_shapes = (
        jax.ShapeDtypeStruct((D, D), jnp.float32),
        jax.ShapeDtypeStruct((1, D), jnp.float32),
        jax.ShapeDtypeStruct((D, D), jnp.float32),
        jax.ShapeDtypeStruct((1, D), jnp.float32),
    )
    return pl.pallas_call(
        _combine_kernel,
        out_shape=out_shapes,
    )(Wk, bk.reshape(1, D), Wv, bv.reshape(1, D), rel_att, rel_msg)


def _qkv_kernel(h_ref, Wq_ref, bq_ref, Mk_ref, bke_ref, Mv_ref, bve_ref,
                q_ref, k_ref, v_ref):
    x = h_ref[...]
    q = jnp.dot(x, Wq_ref[...].T,
                preferred_element_type=jnp.float32) + bq_ref[...]
    k = jnp.dot(x, Mk_ref[...],
                preferred_element_type=jnp.float32) + bke_ref[...]
    v = jnp.dot(x, Mv_ref[...],
                preferred_element_type=jnp.float32) + bve_ref[...]
    q_ref[0] = q[:, :HW]
    q_ref[1] = q[:, HW:]
    k_ref[0] = k[:, :HW]
    k_ref[1] = k[:, HW:]
    v_ref[0] = v[:, :HW]
    v_ref[1] = v[:, HW:]


def _qkv(h, Wq, bq, Mk, bke, Mv, bve):
    grid = (N // ROW_BLK,)
    w_spec = pl.BlockSpec((D, D), lambda i: (0, 0))
    b_spec = pl.BlockSpec((1, D), lambda i: (0, 0))
    out_spec = pl.BlockSpec((NC, ROW_BLK, HW), lambda i: (0, i, 0))
    return pl.pallas_call(
        _qkv_kernel,
        grid=grid,
        in_specs=[pl.BlockSpec((ROW_BLK, D), lambda i: (i, 0)),
                  w_spec, b_spec, w_spec, b_spec, w_spec, b_spec],
        out_specs=(out_spec, out_spec, out_spec),
        out_shape=(jax.ShapeDtypeStruct((NC, N, HW), jnp.float32),) * 3,
    )(h, Wq, bq.reshape(1, D), Mk, bke, Mv, bve)


def _dg(x, idx):
    return x.at[idx].get(mode="promise_in_bounds")


def _edge_body(qt, kt, vt, srch, dsth, scale_h, out,
               src_v0, dst_v0, dsto_v0, src_v1, dst_v1, dsto_v1,
               qb0, qb1, kb, vb, idx40, idx41, exb0, exb1, scale_v,
               sh_agg, sh_den, semq0, semq1, semk, semv, semd0, semd1):
    c = lax.axis_index("c")
    s = lax.axis_index("s")
    lanes = lax.iota(jnp.int32, 16)
    zeros16 = jnp.zeros((16,), jnp.float32)

    src_v = [src_v0, src_v1]
    dst_v = [dst_v0, dst_v1]
    dsto_v = [dsto_v0, dsto_v1]
    qb = [qb0, qb1]
    semq = [semq0, semq1]
    idx4_v = [idx40, idx41]
    exb = [exb0, exb1]
    semd = [semd0, semd1]

    r8 = (lanes + 8) & 15
    p4 = (lanes & 8) | ((lanes + 4) & 7)
    p2 = (lanes & 8) | ((lanes + 2) & 7)
    p1 = (lanes & 8) | ((lanes + 1) & 7)
    i8 = (lanes & 1) * 8
    l4 = (lanes - 4) & 15
    l8 = (lanes - 8) & 15
    l12 = (lanes - 12) & 15

    pltpu.sync_copy(scale_h.at[c], scale_v)
    off = c * N

    @pl.loop(0, RCH)
    def _zero_buf(r):
        for i in range(8):
            qb0.at[r][pl.ds(i * 16, 16)] = zeros16

    @pl.loop(0, C * 4 // 16)
    def _zero_den(t):
        exb0[pl.ds(16 * t, 16)] = zeros16

    for t in range(8):
        cc = s + NS * t

        @pl.when(cc < NRC)
        def _():
            pltpu.sync_copy(qb0, sh_agg.at[pl.ds(cc * RCH, RCH)])
            pltpu.sync_copy(exb0, sh_den.at[pl.ds(cc * RCH * 4, RCH * 4)])

    plsc.subcore_barrier()


    def stage(b, ch):
        base = s * EPT + ch * C
        pltpu.sync_copy(srch.at[pl.ds(base, C)], src_v[b])
        pltpu.sync_copy(dsth.at[pl.ds(base, C)], dst_v[b])
        for g in range(C // 16):
            sl = pl.ds(g * 16, 16)
            src_v[b][sl] = src_v[b][sl] + off
            dsto_v[b][sl] = dst_v[b][sl] + off
        pltpu.async_copy(qt.at[dsto_v[b]], qb[b], semq[b])

    def _attn(b, e):
        qe = qb[b].at[e]
        ke = kb.at[e]
        ss = []
        for hh in range(4):
            pa = qe[pl.ds(hh * 32, 16)] * ke[pl.ds(hh * 32, 16)]
            pb = qe[pl.ds(hh * 32 + 16, 16)] * ke[pl.ds(hh * 32 + 16, 16)]
            ss.append(pa + pb)
        t0 = ss[0] + _dg(ss[0], r8)
        t1 = ss[1] + _dg(ss[1], r8)
        t2 = ss[2] + _dg(ss[2], r8)
        t3 = ss[3] + _dg(ss[3], r8)
        u01 = jnp.where(lanes < 8, t0, _dg(t1, r8))
        u23 = jnp.where(lanes < 8, t2, _dg(t3, r8))
        u01 = u01 + _dg(u01, p4)
        u23 = u23 + _dg(u23, p4)
        u01 = u01 + _dg(u01, p2)
        u23 = u23 + _dg(u23, p2)
        u01 = u01 + _dg(u01, p1)
        u23 = u23 + _dg(u23, p1)
        w = jnp.where(lanes < 2, _dg(u01, i8), _dg(u23, i8))
        sv = scale_v[pl.ds(0, 16)]
        return jnp.exp(w * sv)

    def dot_phase(b):
        @pl.loop(0, C // 4)
        def _quad(p):
            ex0 = _attn(b, 4 * p)
            ex1 = _attn(b, 4 * p + 1)
            ex2 = _attn(b, 4 * p + 2)
            ex3 = _attn(b, 4 * p + 3)
            packed = jnp.where(
                lanes < 4, ex0,
                jnp.where(lanes < 8, _dg(ex1, l4),
                          jnp.where(lanes < 12, _dg(ex2, l8),
                                    _dg(ex3, l12))))
            exb[b][pl.ds(16 * p, 16)] = packed

    def wv_phase(b):
        @pl.loop(0, C // 4)
        def _quad(p):
            exB = exb[b][pl.ds(16 * p, 16)]
            for sub in range(4):
                e = 4 * p + sub
                ve = vb.at[e]
                we = qb[b].at[e]
                for hh in range(4):
                    wb = _dg(exB, jnp.full((16,), 4 * sub + hh, jnp.int32))
                    for j in range(2):
                        sl = pl.ds(hh * 32 + j * 16, 16)
                        we[sl] = ve[sl] * wb

    def scatter(b):
        @pl.loop(0, C // 16)
        def _grp(g4):
            dvv = dst_v[b][pl.ds(g4 * 16, 16)]
            for q in range(4):
                d4 = _dg(dvv, (lanes >> 2) + 4 * q)
                idx4_v[b][pl.ds(g4 * 64 + q * 16, 16)] = (
                    d4 * 4 + (lanes & 3))

        pltpu.sync_copy(qb[b], sh_agg.at[dst_v[b]], add=True)
        pltpu.async_copy(exb[b], sh_den.at[idx4_v[b]], semd[b], add=True)

    def wait_den(b):
        pltpu.make_async_copy(exb[b], sh_den.at[idx4_v[b]], semd[b]).wait()

    def substep(b, ch, first, last):
        if not first:
            wait_den(b)
        if not last:
            stage(1 - b, ch + 1)
        pltpu.make_async_copy(qt.at[dsto_v[b]], qb[b], semq[b]).wait()
        pltpu.make_async_copy(kt.at[src_v[b]], kb, semk).wait()
        dot_phase(b)
        if not last:
            pltpu.async_copy(kt.at[src_v[1 - b]], kb, semk)
        pltpu.make_async_copy(vt.at[src_v[b]], vb, semv).wait()
        wv_phase(b)
        if not last:
            pltpu.async_copy(vt.at[src_v[1 - b]], vb, semv)
        scatter(b)

    stage(0, 0)
    pltpu.async_copy(kt.at[src_v[0]], kb, semk)
    pltpu.async_copy(vt.at[src_v[0]], vb, semv)

    substep(0, 0, True, False)
    substep(1, 1, True, False)

    @pl.loop(1, (NCHUNK - 1) // 2)
    def _super(ssi):
        substep(0, 2 * ssi, False, False)
        substep(1, 2 * ssi + 1, False, False)

    substep(0, NCHUNK - 1, False, True)
    wait_den(0)
    wait_den(1)

    plsc.subcore_barrier()

    for t in range(8):
        cc = s + NS * t

        @pl.when(cc < NRC)
        def _():
            r0 = cc * RCH
            pltpu.sync_copy(sh_agg.at[pl.ds(r0, RCH)], qb0)
            pltpu.sync_copy(sh_den.at[pl.ds(r0 * 4, RCH * 4)], exb0)

            @pl.loop(0, RCH // 4)
            def _rowquad(p):
                den = exb0[pl.ds(16 * p, 16)]
                rcp = 1.0 / (den + 1e-16)
                for rr in range(4):
                    zr = qb0.at[4 * p + rr]
                    for hh in range(4):
                        rb = _dg(rcp, jnp.full((16,), rr * 4 + hh,
                                               jnp.int32))
                        for j in range(2):
                            sl = pl.ds(hh * 32 + j * 16, 16)
                            zr[sl] = zr[sl] * rb

            pltpu.sync_copy(qb0, out.at[pl.ds(c * N + r0, RCH)])


def _edge_sc(qt, kt, vt, src, dst, scale):
    mesh = plsc.VectorSubcoreMesh(core_axis_name="c", subcore_axis_name="s")

    f = pl.kernel(
        _edge_body,
        out_type=jax.ShapeDtypeStruct((NC * N, HW), jnp.float32),
        mesh=mesh,
        scratch_types=(
            [pltpu.VMEM((C,), jnp.int32)] * 6
            + [pltpu.VMEM((C, HW), jnp.float32)] * 4
            + [pltpu.VMEM((C * 4,), jnp.int32)] * 2
            + [pltpu.VMEM((C * 4,), jnp.float32)] * 2
            + [pltpu.VMEM((16,), jnp.float32),
               pltpu.VMEM_SHARED((N, HW), jnp.float32),
               pltpu.VMEM_SHARED((N * 4,), jnp.float32)]
            + [pltpu.SemaphoreType.DMA] * 6
        ),
    )
    return f(qt.reshape(NC * N, HW), kt.reshape(NC * N, HW),
             vt.reshape(NC * N, HW), src, dst, scale)


def _final_kernel(agg_ref, h_ref, Wa_ref, ba_ref, skip_ref, out_ref):
    alpha = jax.nn.sigmoid(skip_ref[0])
    dn = (((1,), (1,)), ((), ()))
    trans = lax.dot_general(agg_ref[0], Wa_ref[:, :HW], dn,
                            preferred_element_type=jnp.float32)
    trans = trans + lax.dot_general(agg_ref[1], Wa_ref[:, HW:], dn,
                                    preferred_element_type=jnp.float32)
    trans = trans + ba_ref[...]
    out_ref[...] = trans * alpha + h_ref[...] * (1.0 - alpha)


def _final(agg2, h, Wa, ba, skip):
    grid = (N // ROW_BLK,)
    return pl.pallas_call(
        _final_kernel,
        grid=grid,
        in_specs=[pl.BlockSpec((NC, ROW_BLK, HW), lambda i: (0, i, 0)),
                  pl.BlockSpec((ROW_BLK, D), lambda i: (i, 0)),
                  pl.BlockSpec((D, D), lambda i: (0, 0)),
                  pl.BlockSpec((1, D), lambda i: (0, 0)),
                  pl.BlockSpec(memory_space=pltpu.SMEM)],
        out_specs=pl.BlockSpec((ROW_BLK, D), lambda i: (i, 0)),
        out_shape=jax.ShapeDtypeStruct((N, D), jnp.float32),
    )(agg2, h, Wa, ba.reshape(1, D), skip)


def kernel(h, edge_index, Wk, bk, Wq, bq, Wv, bv, Wa, ba, rel_pri, rel_att,
           rel_msg, skip):
    src = edge_index[0]
    dst = edge_index[1]
    Mk, bke, Mv, bve = _combine(Wk, bk, Wv, bv, rel_att, rel_msg)
    qt, kt, vt = _qkv(h, Wq, bq, Mk, bke, Mv, bve)
    scale = jnp.zeros((NC, 16), jnp.float32)
    scale = scale.at[:, :4].set(
        (rel_pri / math.sqrt(DK)).reshape(NC, 4))
    agg = _edge_sc(qt, kt, vt, src, dst, scale)
    return _final(agg.reshape(NC, N, HW), h, Wa, ba, skip)


# speedup vs baseline: 55.9420x; 1.2260x over previous
import math
import functools

import jax
import jax.numpy as jnp
from jax import lax
from jax.experimental import pallas as pl
from jax.experimental.pallas import tpu as pltpu
from jax.experimental.pallas import tpu_sc as plsc

N = 10000
E = 160000
D = 256
H = 8
DK = 32
ROW_BLK = 1000

NS = 16
NC = 2
EPT = E // NS
C = 80
NCHUNK = EPT // C
RCH = 80
NRC = N // RCH
HW = 128


def _combine_kernel(Wk_ref, bk_ref, Wv_ref, bv_ref, ra_ref, rm_ref,
                    Mk_ref, bke_ref, Mv_ref, bve_ref):
    for h in range(H):
        wk = Wk_ref[pl.ds(h * DK, DK), :]
        wv = Wv_ref[pl.ds(h * DK, DK), :]
        ra = ra_ref[h]
        rm = rm_ref[h]
        dn = (((0,), (0,)), ((), ()))
        Mk_ref[:, pl.ds(h * DK, DK)] = lax.dot_general(
            wk, ra, dn, preferred_element_type=jnp.float32)
        Mv_ref[:, pl.ds(h * DK, DK)] = lax.dot_general(
            wv, rm, dn, preferred_element_type=jnp.float32)
        bke_ref[0, pl.ds(h * DK, DK)] = bk_ref[0, pl.ds(h * DK, DK)] @ ra
        bve_ref[0, pl.ds(h * DK, DK)] = bv_ref[0, pl.ds(h * DK, DK)] @ rm


def _combine(Wk, bk, Wv, bv, rel_att, rel_msg):
    out_shapes = (
        jax.ShapeDtypeStruct((D, D), jnp.float32),
        jax.ShapeDtypeStruct((1, D), jnp.float32),
        jax.ShapeDtypeStruct((D, D), jnp.float32),
        jax.ShapeDtypeStruct((1, D), jnp.float32),
    )
    return pl.pallas_call(
        _combine_kernel,
        out_shape=out_shapes,
    )(Wk, bk.reshape(1, D), Wv, bv.reshape(1, D), rel_att, rel_msg)


def _qkv_kernel(h_ref, Wq_ref, bq_ref, Mk_ref, bke_ref, Mv_ref, bve_ref,
                q_ref, k_ref, v_ref):
    x = h_ref[...]
    q = jnp.dot(x, Wq_ref[...].T,
                preferred_element_type=jnp.float32) + bq_ref[...]
    k = jnp.dot(x, Mk_ref[...],
                preferred_element_type=jnp.float32) + bke_ref[...]
    v = jnp.dot(x, Mv_ref[...],
                preferred_element_type=jnp.float32) + bve_ref[...]
    q_ref[0] = q[:, :HW]
    q_ref[1] = q[:, HW:]
    k_ref[0] = k[:, :HW]
    k_ref[1] = k[:, HW:]
    v_ref[0] = v[:, :HW]
    v_ref[1] = v[:, HW:]


def _qkv(h, Wq, bq, Mk, bke, Mv, bve):
    grid = (N // ROW_BLK,)
    w_spec = pl.BlockSpec((D, D), lambda i: (0, 0))
    b_spec = pl.BlockSpec((1, D), lambda i: (0, 0))
    out_spec = pl.BlockSpec((NC, ROW_BLK, HW), lambda i: (0, i, 0))
    return pl.pallas_call(
        _qkv_kernel,
        grid=grid,
        in_specs=[pl.BlockSpec((ROW_BLK, D), lambda i: (i, 0)),
                  w_spec, b_spec, w_spec, b_spec, w_spec, b_spec],
        out_specs=(out_spec, out_spec, out_spec),
        out_shape=(jax.ShapeDtypeStruct((NC, N, HW), jnp.float32),) * 3,
    )(h, Wq, bq.reshape(1, D), Mk, bke, Mv, bve)


def _dg(x, idx):
    return x.at[idx].get(mode="promise_in_bounds")


def _edge_body(qt, kt, vt, srch, dsth, scale_h, out,
               src_v0, dst_v0, dsto_v0, src_v1, dst_v1, dsto_v1,
               qb0, qb1, kb, vb, idx40, idx41, exb0, exb1, scale_v,
               psrc0, pdst0, psrc1, pdst1,
               sh_agg, sh_den, semq0, semq1, semk, semv, semd0, semd1,
               semp):
    c = lax.axis_index("c")
    s = lax.axis_index("s")
    lanes = lax.iota(jnp.int32, 16)
    zeros16 = jnp.zeros((16,), jnp.float32)

    src_v = [src_v0, src_v1]
    dst_v = [dst_v0, dst_v1]
    dsto_v = [dsto_v0, dsto_v1]
    qb = [qb0, qb1]
    semq = [semq0, semq1]
    idx4_v = [idx40, idx41]
    exb = [exb0, exb1]
    semd = [semd0, semd1]
    psrc = [psrc0, psrc1]
    pdst = [pdst0, pdst1]

    r8 = (lanes + 8) & 15
    p4 = (lanes & 8) | ((lanes + 4) & 7)
    p2 = (lanes & 8) | ((lanes + 2) & 7)
    p1 = (lanes & 8) | ((lanes + 1) & 7)
    i8 = (lanes & 1) * 8
    l4 = (lanes - 4) & 15
    l8 = (lanes - 8) & 15
    l12 = (lanes - 12) & 15

    pltpu.sync_copy(scale_h.at[c], scale_v)
    off = c * N

    @pl.loop(0, RCH)
    def _zero_buf(r):
        for i in range(8):
            qb0.at[r][pl.ds(i * 16, 16)] = zeros16

    @pl.loop(0, C * 4 // 16)
    def _zero_den(t):
        exb0[pl.ds(16 * t, 16)] = zeros16

    for t in range(8):
        cc = s + NS * t

        @pl.when(cc < NRC)
        def _():
            pltpu.sync_copy(qb0, sh_agg.at[pl.ds(cc * RCH, RCH)])
            pltpu.sync_copy(exb0, sh_den.at[pl.ds(cc * RCH * 4, RCH * 4)])

    plsc.subcore_barrier()


    def fire_pair(pp, p):
        base = s * EPT + p * 2 * C
        pltpu.async_copy(srch.at[pl.ds(base, 2 * C)], psrc[pp], semp)
        pltpu.async_copy(dsth.at[pl.ds(base, 2 * C)], pdst[pp], semp)

    def wait_pair(pp):
        pltpu.make_async_copy(srch.at[pl.ds(0, 2 * C)], psrc[pp],
                              semp).wait()
        pltpu.make_async_copy(dsth.at[pl.ds(0, 2 * C)], pdst[pp],
                              semp).wait()

    def build_from_pair(b, pp, half):
        for g in range(C // 16):
            sl = pl.ds(g * 16, 16)
            sp = pl.ds(half * C + g * 16, 16)
            raw_d = pdst[pp][sp]
            dst_v[b][sl] = raw_d
            dsto_v[b][sl] = raw_d + off
            src_v[b][sl] = psrc[pp][sp] + off

    def stage(b, ch):
        half = lax.rem(ch, 2)
        p = lax.div(ch, 2)
        pp_dyn = lax.rem(p, 2)

        can_fire = p < (NCHUNK - 1) // 2

        @pl.when(jnp.logical_and(half == 0, pp_dyn == 0))
        def _():
            wait_pair(0)
            build_from_pair(b, 0, 0)

        @pl.when(jnp.logical_and(jnp.logical_and(half == 0, pp_dyn == 0),
                                 can_fire))
        def _():
            fire_pair(1, p + 1)

        @pl.when(jnp.logical_and(half == 0, pp_dyn == 1))
        def _():
            wait_pair(1)
            build_from_pair(b, 1, 0)

        @pl.when(jnp.logical_and(jnp.logical_and(half == 0, pp_dyn == 1),
                                 can_fire))
        def _():
            fire_pair(0, p + 1)

        @pl.when(jnp.logical_and(half == 1, pp_dyn == 0))
        def _():
            build_from_pair(b, 0, 1)

        @pl.when(jnp.logical_and(half == 1, pp_dyn == 1))
        def _():
            build_from_pair(b, 1, 1)

        pltpu.async_copy(qt.at[dsto_v[b]], qb[b], semq[b])

    def _attn(b, e):
        qe = qb[b].at[e]
        ke = kb.at[e]
        ss = []
        for hh in range(4):
            pa = qe[pl.ds(hh * 32, 16)] * ke[pl.ds(hh * 32, 16)]
            pb = qe[pl.ds(hh * 32 + 16, 16)] * ke[pl.ds(hh * 32 + 16, 16)]
            ss.append(pa + pb)
        t0 = ss[0] + _dg(ss[0], r8)
        t1 = ss[1] + _dg(ss[1], r8)
        t2 = ss[2] + _dg(ss[2], r8)
        t3 = ss[3] + _dg(ss[3], r8)
        u01 = jnp.where(lanes < 8, t0, _dg(t1, r8))
        u23 = jnp.where(lanes < 8, t2, _dg(t3, r8))
        u01 = u01 + _dg(u01, p4)
        u23 = u23 + _dg(u23, p4)
        u01 = u01 + _dg(u01, p2)
        u23 = u23 + _dg(u23, p2)
        u01 = u01 + _dg(u01, p1)
        u23 = u23 + _dg(u23, p1)
        w = jnp.where(lanes < 2, _dg(u01, i8), _dg(u23, i8))
        sv = scale_v[pl.ds(0, 16)]
        return jnp.exp(w * sv)

    def dot_phase(b):
        @pl.loop(0, C // 4)
        def _quad(p):
            ex0 = _attn(b, 4 * p)
            ex1 = _attn(b, 4 * p + 1)
            ex2 = _attn(b, 4 * p + 2)
            ex3 = _attn(b, 4 * p + 3)
            packed = jnp.where(
                lanes < 4, ex0,
                jnp.where(lanes < 8, _dg(ex1, l4),
                          jnp.where(lanes < 12, _dg(ex2, l8),
                                    _dg(ex3, l12))))
            exb[b][pl.ds(16 * p, 16)] = packed

    def wv_phase(b):
        @pl.loop(0, C // 4)
        def _quad(p):
            exB = exb[b][pl.ds(16 * p, 16)]
            for sub in range(4):
                e = 4 * p + sub
                ve = vb.at[e]
                we = qb[b].at[e]
                for hh in range(4):
                    wb = _dg(exB, jnp.full((16,), 4 * sub + hh, jnp.int32))
                    for j in range(2):
                        sl = pl.ds(hh * 32 + j * 16, 16)
                        we[sl] = ve[sl] * wb

    def scatter(b):
        @pl.loop(0, C // 16)
        def _grp(g4):
            dvv = dst_v[b][pl.ds(g4 * 16, 16)]
            for q in range(4):
                d4 = _dg(dvv, (lanes >> 2) + 4 * q)
                idx4_v[b][pl.ds(g4 * 64 + q * 16, 16)] = (
                    d4 * 4 + (lanes & 3))

        pltpu.sync_copy(qb[b], sh_agg.at[dst_v[b]], add=True)
        pltpu.async_copy(exb[b], sh_den.at[idx4_v[b]], semd[b], add=True)

    def wait_den(b):
        pltpu.make_async_copy(exb[b], sh_den.at[idx4_v[b]], semd[b]).wait()

    def substep(b, ch, first, last):
        if not first:
            wait_den(b)
        if not last:
            stage(1 - b, ch + 1)
        pltpu.make_async_copy(qt.at[dsto_v[b]], qb[b], semq[b]).wait()
        pltpu.make_async_copy(kt.at[src_v[b]], kb, semk).wait()
        dot_phase(b)
        if not last:
            pltpu.async_copy(kt.at[src_v[1 - b]], kb, semk)
        pltpu.make_async_copy(vt.at[src_v[b]], vb, semv).wait()
        wv_phase(b)
        if not last:
            pltpu.async_copy(vt.at[src_v[1 - b]], vb, semv)
        scatter(b)

    fire_pair(0, 0)
    wait_pair(0)
    build_from_pair(0, 0, 0)
    fire_pair(1, 1)
    pltpu.async_copy(qt.at[dsto_v[0]], qb[0], semq[0])
    pltpu.async_copy(kt.at[src_v[0]], kb, semk)
    pltpu.async_copy(vt.at[src_v[0]], vb, semv)

    substep(0, 0, True, False)
    substep(1, 1, True, False)

    @pl.loop(1, (NCHUNK - 1) // 2)
    def _super(ssi):
        substep(0, 2 * ssi, False, False)
        substep(1, 2 * ssi + 1, False, False)

    substep(0, NCHUNK - 1, False, True)
    wait_den(0)
    wait_den(1)

    plsc.subcore_barrier()

    for t in range(8):
        cc = s + NS * t

        @pl.when(cc < NRC)
        def _():
            r0 = cc * RCH
            pltpu.sync_copy(sh_agg.at[pl.ds(r0, RCH)], qb0)
            pltpu.sync_copy(sh_den.at[pl.ds(r0 * 4, RCH * 4)], exb0)

            @pl.loop(0, RCH // 4)
            def _rowquad(p):
                den = exb0[pl.ds(16 * p, 16)]
                rcp = 1.0 / (den + 1e-16)
                for rr in range(4):
                    zr = qb0.at[4 * p + rr]
                    for hh in range(4):
                        rb = _dg(rcp, jnp.full((16,), rr * 4 + hh,
                                               jnp.int32))
                        for j in range(2):
                            sl = pl.ds(hh * 32 + j * 16, 16)
                            zr[sl] = zr[sl] * rb

            pltpu.sync_copy(qb0, out.at[pl.ds(c * N + r0, RCH)])


def _edge_sc(qt, kt, vt, src, dst, scale):
    mesh = plsc.VectorSubcoreMesh(core_axis_name="c", subcore_axis_name="s")

    f = pl.kernel(
        _edge_body,
        out_type=jax.ShapeDtypeStruct((NC * N, HW), jnp.float32),
        mesh=mesh,
        scratch_types=(
            [pltpu.VMEM((C,), jnp.int32)] * 6
            + [pltpu.VMEM((C, HW), jnp.float32)] * 4
            + [pltpu.VMEM((C * 4,), jnp.int32)] * 2
            + [pltpu.VMEM((C * 4,), jnp.float32)] * 2
            + [pltpu.VMEM((16,), jnp.float32)]
            + [pltpu.VMEM((2 * C,), jnp.int32)] * 4
            + [pltpu.VMEM_SHARED((N, HW), jnp.float32),
               pltpu.VMEM_SHARED((N * 4,), jnp.float32)]
            + [pltpu.SemaphoreType.DMA] * 7
        ),
    )
    return f(qt.reshape(NC * N, HW), kt.reshape(NC * N, HW),
             vt.reshape(NC * N, HW), src, dst, scale)


def _final_kernel(agg_ref, h_ref, Wa_ref, ba_ref, skip_ref, out_ref):
    alpha = jax.nn.sigmoid(skip_ref[0])
    dn = (((1,), (1,)), ((), ()))
    trans = lax.dot_general(agg_ref[0], Wa_ref[:, :HW], dn,
                            preferred_element_type=jnp.float32)
    trans = trans + lax.dot_general(agg_ref[1], Wa_ref[:, HW:], dn,
                                    preferred_element_type=jnp.float32)
    trans = trans + ba_ref[...]
    out_ref[...] = trans * alpha + h_ref[...] * (1.0 - alpha)


def _final(agg2, h, Wa, ba, skip):
    grid = (N // ROW_BLK,)
    return pl.pallas_call(
        _final_kernel,
        grid=grid,
        in_specs=[pl.BlockSpec((NC, ROW_BLK, HW), lambda i: (0, i, 0)),
                  pl.BlockSpec((ROW_BLK, D), lambda i: (i, 0)),
                  pl.BlockSpec((D, D), lambda i: (0, 0)),
                  pl.BlockSpec((1, D), lambda i: (0, 0)),
                  pl.BlockSpec(memory_space=pltpu.SMEM)],
        out_specs=pl.BlockSpec((ROW_BLK, D), lambda i: (i, 0)),
        out_shape=jax.ShapeDtypeStruct((N, D), jnp.float32),
    )(agg2, h, Wa, ba.reshape(1, D), skip)


def kernel(h, edge_index, Wk, bk, Wq, bq, Wv, bv, Wa, ba, rel_pri, rel_att,
           rel_msg, skip):
    ei = jnp.pad(edge_index, ((0, 0), (0, 2 * C)))
    src = ei[0]
    dst = ei[1]
    Mk, bke, Mv, bve = _combine(Wk, bk, Wv, bv, rel_att, rel_msg)
    qt, kt, vt = _qkv(h, Wq, bq, Mk, bke, Mv, bve)
    scale = jnp.zeros((NC, 16), jnp.float32)
    scale = scale.at[:, :4].set(
        (rel_pri / math.sqrt(DK)).reshape(NC, 4))
    agg = _edge_sc(qt, kt, vt, src, dst, scale)
    return _final(agg.reshape(NC, N, HW), h, Wa, ba, skip)
